# baseline (speedup 1.0000x reference)
import functools

import jax
import jax.numpy as jnp
from jax import lax
from jax.experimental import pallas as pl
from jax.experimental.pallas import tpu as pltpu
from jax.experimental.pallas import tpu_sc as plsc

D = 50
DP = 56
NW = 32
BLK = 128
CH = 6400


@functools.partial(jax.jit, static_argnums=(0, 1))
def _action_embed(N, V, table, type_flat, value_flat):
    n_per_w = N // NW
    nchunk = n_per_w // CH
    nb = CH // BLK
    mesh = plsc.VectorSubcoreMesh(core_axis_name="c", subcore_axis_name="s")

    @functools.partial(
        pl.kernel,
        mesh=mesh,
        compiler_params=pltpu.CompilerParams(use_tc_tiling_on_sc=False),
        out_type=jax.ShapeDtypeStruct((N, DP), jnp.float32),
        scratch_types=[
            pltpu.VMEM((CH,), jnp.int32),
            pltpu.VMEM((CH,), jnp.int32),
            pltpu.VMEM((nb, BLK), jnp.int32),
            pltpu.VMEM((BLK, DP), jnp.float32),
            pltpu.VMEM((BLK, DP), jnp.float32),
            pltpu.SemaphoreType.DMA,
            pltpu.SemaphoreType.DMA,
        ],
    )
    def k(table_h, type_h, value_h, out_h, t_v, v_v, idx_v, rows_a, rows_b,
          sem_a, sem_b):
        wid = lax.axis_index("s") * 2 + lax.axis_index("c")
        base_w = wid * n_per_w

        for c in range(nchunk):
            base = base_w + c * CH
            pltpu.sync_copy(type_h.at[pl.ds(base, CH)], t_v)
            pltpu.sync_copy(value_h.at[pl.ds(base, CH)], v_v)

            def idx_body(j, _):
                t = t_v[pl.ds(j * 16, 16)]
                v = v_v[pl.ds(j * 16, 16)]
                b = j // (BLK // 16)
                col = (j % (BLK // 16)) * 16
                idx_v[b, pl.ds(col, 16)] = v + t * V
                return 0

            lax.fori_loop(0, CH // 16, idx_body, 0)

            pltpu.async_copy(table_h.at[idx_v.at[0]], rows_a, sem_a)

            def pair_body(g, _):
                pltpu.async_copy(
                    table_h.at[idx_v.at[2 * g + 1]], rows_b, sem_b)
                pltpu.make_async_copy(
                    table_h.at[idx_v.at[0]], rows_a, sem_a).wait()
                pltpu.sync_copy(
                    rows_a, out_h.at[pl.ds(base + (2 * g) * BLK, BLK)])
                pltpu.async_copy(
                    table_h.at[idx_v.at[(2 * g + 2) % nb]], rows_a, sem_a)
                pltpu.make_async_copy(
                    table_h.at[idx_v.at[0]], rows_b, sem_b).wait()
                pltpu.sync_copy(
                    rows_b, out_h.at[pl.ds(base + (2 * g + 1) * BLK, BLK)])
                return 0

            lax.fori_loop(0, nb // 2, pair_body, 0)
            pltpu.make_async_copy(table_h.at[idx_v.at[0]], rows_a, sem_a).wait()

    return k(table, type_flat, value_flat)


def kernel(action, rule_table, token_table):
    V = rule_table.shape[0]
    _, B, L = action.shape
    N = B * L
    table = jnp.pad(
        jnp.concatenate([rule_table, token_table], axis=0),
        ((0, 0), (0, DP - D)),
    )
    type_flat = action[0].reshape(N)
    value_flat = action[1].reshape(N)
    out = _action_embed(N, V, table, type_flat, value_flat)
    return out.reshape(B, L, DP)[:, :, :D]


# speedup vs baseline: 2.3606x; 2.3606x over previous
import functools

import jax
import jax.numpy as jnp
from jax import lax
from jax.experimental import pallas as pl
from jax.experimental.pallas import tpu as pltpu
from jax.experimental.pallas import tpu_sc as plsc

D = 50
DP = 56
NW = 32
BLK = 400
CH = 6400


@functools.partial(jax.jit, static_argnums=(0, 1, 2))
def _action_embed(B, L, V, table, type_flat, value_flat):
    N = B * L
    n_per_w = N // NW
    nchunk = n_per_w // CH
    nb = CH // BLK
    bpb = BLK // L
    mesh = plsc.VectorSubcoreMesh(core_axis_name="c", subcore_axis_name="s")

    @functools.partial(
        pl.kernel,
        mesh=mesh,
        compiler_params=pltpu.CompilerParams(use_tc_tiling_on_sc=False),
        out_type=jax.ShapeDtypeStruct((B, L, DP), jnp.float32),
        scratch_types=[
            pltpu.VMEM((CH,), jnp.int32),
            pltpu.VMEM((CH,), jnp.int32),
            pltpu.VMEM((CH,), jnp.int32),
            pltpu.VMEM((BLK, DP), jnp.float32),
            pltpu.VMEM((BLK, DP), jnp.float32),
            pltpu.SemaphoreType.DMA,
            pltpu.SemaphoreType.DMA,
        ],
    )
    def k(table_h, type_h, value_h, out_h, t_v, v_v, idx_v, rows_a, rows_b,
          sem_a, sem_b):
        wid = lax.axis_index("s") * 2 + lax.axis_index("c")
        base_w = wid * n_per_w

        def writes(rows_v, b0):
            for i in range(bpb):
                pltpu.sync_copy(rows_v.at[pl.ds(L * i, L)], out_h.at[b0 + i])

        def gather(q, rows_v, sem):
            pltpu.async_copy(
                table_h.at[idx_v.at[pl.ds(q * BLK, BLK)]], rows_v, sem)

        for c in range(nchunk):
            base = base_w + c * CH
            b_c = base // L
            pltpu.sync_copy(type_h.at[pl.ds(base, CH)], t_v)
            pltpu.sync_copy(value_h.at[pl.ds(base, CH)], v_v)

            def idx_body(j, _):
                t = t_v[pl.ds(j * 16, 16)]
                v = v_v[pl.ds(j * 16, 16)]
                idx_v[pl.ds(j * 16, 16)] = v + t * V
                return 0

            lax.fori_loop(0, CH // 16, idx_body, 0)

            gather(0, rows_a, sem_a)

            def pair_body(g, _):
                gather(2 * g + 1, rows_b, sem_b)
                pltpu.make_async_copy(
                    table_h.at[idx_v.at[pl.ds(0, BLK)]], rows_a, sem_a).wait()
                writes(rows_a, b_c + (2 * g) * bpb)
                gather((2 * g + 2) % nb, rows_a, sem_a)
                pltpu.make_async_copy(
                    table_h.at[idx_v.at[pl.ds(0, BLK)]], rows_b, sem_b).wait()
                writes(rows_b, b_c + (2 * g + 1) * bpb)
                return 0

            lax.fori_loop(0, nb // 2, pair_body, 0)
            pltpu.make_async_copy(
                table_h.at[idx_v.at[pl.ds(0, BLK)]], rows_a, sem_a).wait()

    return k(table, type_flat, value_flat)


def kernel(action, rule_table, token_table):
    V = rule_table.shape[0]
    _, B, L = action.shape
    N = B * L
    table = jnp.pad(
        jnp.concatenate([rule_table, token_table], axis=0),
        ((0, 0), (0, DP - D)),
    )
    type_flat = action[0].reshape(N)
    value_flat = action[1].reshape(N)
    out = _action_embed(B, L, V, table, type_flat, value_flat)
    return out[:, :, :D]


# speedup vs baseline: 2.5994x; 1.1011x over previous
import functools

import jax
import jax.numpy as jnp
from jax import lax
from jax.experimental import pallas as pl
from jax.experimental.pallas import tpu as pltpu
from jax.experimental.pallas import tpu_sc as plsc

D = 50
DP = 56
NW = 32
BLK = 400
CH = 6400


def _pad_stack_tables(rule_table, token_table):
    V = rule_table.shape[0]
    R = 4000
    grid = V // R

    def body(r_ref, t_ref, o_ref):
        z = jnp.zeros((R, DP - D), jnp.float32)
        o_ref[0] = jnp.concatenate([r_ref[...], z], axis=1)
        o_ref[1] = jnp.concatenate([t_ref[...], z], axis=1)

    out = pl.pallas_call(
        body,
        grid=(grid,),
        in_specs=[
            pl.BlockSpec((R, D), lambda i: (i, 0)),
            pl.BlockSpec((R, D), lambda i: (i, 0)),
        ],
        out_specs=pl.BlockSpec((2, R, DP), lambda i: (0, i, 0)),
        out_shape=jax.ShapeDtypeStruct((2, V, DP), jnp.float32),
    )(rule_table, token_table)
    return out.reshape(2 * V, DP)


@functools.partial(jax.jit, static_argnums=(0, 1, 2))
def _action_embed(B, L, V, table, action2):
    N = B * L
    n_per_w = N // NW
    nchunk = n_per_w // CH
    nb = CH // BLK
    bpb = BLK // L
    mesh = plsc.VectorSubcoreMesh(core_axis_name="c", subcore_axis_name="s")

    @functools.partial(
        pl.kernel,
        mesh=mesh,
        compiler_params=pltpu.CompilerParams(use_tc_tiling_on_sc=False),
        out_type=jax.ShapeDtypeStruct((B, L, DP), jnp.float32),
        scratch_types=[
            pltpu.VMEM((CH,), jnp.int32),
            pltpu.VMEM((CH,), jnp.int32),
            pltpu.VMEM((CH,), jnp.int32),
            pltpu.VMEM((BLK, DP), jnp.float32),
            pltpu.VMEM((BLK, DP), jnp.float32),
            pltpu.SemaphoreType.DMA,
            pltpu.SemaphoreType.DMA,
        ],
    )
    def k(table_h, action_h, out_h, t_v, v_v, idx_v, rows_a, rows_b,
          sem_a, sem_b):
        wid = lax.axis_index("s") * 2 + lax.axis_index("c")
        base_w = wid * n_per_w

        def writes(rows_v, b0):
            for i in range(bpb):
                pltpu.sync_copy(rows_v.at[pl.ds(L * i, L)], out_h.at[b0 + i])

        def gather(q, rows_v, sem):
            pltpu.async_copy(
                table_h.at[idx_v.at[pl.ds(q * BLK, BLK)]], rows_v, sem)

        for c in range(nchunk):
            base = base_w + c * CH
            b_c = base // L
            pltpu.sync_copy(action_h.at[0, pl.ds(base, CH)], t_v)
            pltpu.sync_copy(action_h.at[1, pl.ds(base, CH)], v_v)

            def idx_body(j, _):
                t = t_v[pl.ds(j * 16, 16)]
                v = v_v[pl.ds(j * 16, 16)]
                idx_v[pl.ds(j * 16, 16)] = v + t * V
                return 0

            lax.fori_loop(0, CH // 16, idx_body, 0)

            gather(0, rows_a, sem_a)

            def pair_body(g, _):
                gather(2 * g + 1, rows_b, sem_b)
                pltpu.make_async_copy(
                    table_h.at[idx_v.at[pl.ds(0, BLK)]], rows_a, sem_a).wait()
                writes(rows_a, b_c + (2 * g) * bpb)
                gather((2 * g + 2) % nb, rows_a, sem_a)
                pltpu.make_async_copy(
                    table_h.at[idx_v.at[pl.ds(0, BLK)]], rows_b, sem_b).wait()
                writes(rows_b, b_c + (2 * g + 1) * bpb)
                return 0

            lax.fori_loop(0, nb // 2, pair_body, 0)
            pltpu.make_async_copy(
                table_h.at[idx_v.at[pl.ds(0, BLK)]], rows_a, sem_a).wait()

    return k(table, action2)


def kernel(action, rule_table, token_table):
    V = rule_table.shape[0]
    _, B, L = action.shape
    N = B * L
    table = _pad_stack_tables(rule_table, token_table)
    action2 = action.reshape(2, N)
    out = _action_embed(B, L, V, table, action2)
    return out[:, :, :D]


# speedup vs baseline: 3.8199x; 1.4696x over previous
import functools

import jax
import jax.numpy as jnp
from jax import lax
from jax.experimental import pallas as pl
from jax.experimental.pallas import tpu as pltpu
from jax.experimental.pallas import tpu_sc as plsc

D = 50
DP = 128
NW = 32
BLK = 400
CH = 6400


def _pad_stack_tables(rule_table, token_table):
    V = rule_table.shape[0]
    R = 4000
    grid = V // R

    def body(r_ref, t_ref, o_ref):
        z = jnp.zeros((R, DP - D), jnp.float32)
        o_ref[0] = jnp.concatenate([r_ref[...], z], axis=1)
        o_ref[1] = jnp.concatenate([t_ref[...], z], axis=1)

    out = pl.pallas_call(
        body,
        grid=(grid,),
        in_specs=[
            pl.BlockSpec((R, D), lambda i: (i, 0)),
            pl.BlockSpec((R, D), lambda i: (i, 0)),
        ],
        out_specs=pl.BlockSpec((2, R, DP), lambda i: (0, i, 0)),
        out_shape=jax.ShapeDtypeStruct((2, V, DP), jnp.float32),
    )(rule_table, token_table)
    return out.reshape(2 * V, DP)


@functools.partial(jax.jit, static_argnums=(0, 1, 2))
def _action_embed(B, L, V, table, action2):
    N = B * L
    n_per_w = N // NW
    nchunk = n_per_w // CH
    nb = CH // BLK
    bpb = BLK // L
    mesh = plsc.VectorSubcoreMesh(core_axis_name="c", subcore_axis_name="s")

    @functools.partial(
        pl.kernel,
        mesh=mesh,
        compiler_params=pltpu.CompilerParams(use_tc_tiling_on_sc=False),
        out_type=jax.ShapeDtypeStruct((B, 56, DP), jnp.float32),
        scratch_types=[
            pltpu.VMEM((CH,), jnp.int32),
            pltpu.VMEM((CH,), jnp.int32),
            pltpu.VMEM((CH,), jnp.int32),
            pltpu.VMEM((BLK, DP), jnp.float32),
            pltpu.VMEM((BLK, DP), jnp.float32),
            pltpu.SemaphoreType.DMA,
            pltpu.SemaphoreType.DMA,
        ],
    )
    def k(table_h, action_h, out_h, t_v, v_v, idx_v, rows_a, rows_b,
          sem_a, sem_b):
        wid = lax.axis_index("s") * 2 + lax.axis_index("c")
        base_w = wid * n_per_w

        def writes(rows_v, b0):
            for i in range(bpb):
                pltpu.sync_copy(rows_v.at[pl.ds(L * i, L)],
                                out_h.at[b0 + i, pl.ds(0, L)])

        def gather(q, rows_v, sem):
            pltpu.async_copy(
                table_h.at[idx_v.at[pl.ds(q * BLK, BLK)]], rows_v, sem)

        for c in range(nchunk):
            base = base_w + c * CH
            b_c = base // L
            pltpu.sync_copy(action_h.at[0, pl.ds(base, CH)], t_v)
            pltpu.sync_copy(action_h.at[1, pl.ds(base, CH)], v_v)

            def idx_body(j, _):
                t = t_v[pl.ds(j * 16, 16)]
                v = v_v[pl.ds(j * 16, 16)]
                idx_v[pl.ds(j * 16, 16)] = v + t * V
                return 0

            lax.fori_loop(0, CH // 16, idx_body, 0)

            gather(0, rows_a, sem_a)

            def pair_body(g, _):
                gather(2 * g + 1, rows_b, sem_b)
                pltpu.make_async_copy(
                    table_h.at[idx_v.at[pl.ds(0, BLK)]], rows_a, sem_a).wait()
                writes(rows_a, b_c + (2 * g) * bpb)
                gather((2 * g + 2) % nb, rows_a, sem_a)
                pltpu.make_async_copy(
                    table_h.at[idx_v.at[pl.ds(0, BLK)]], rows_b, sem_b).wait()
                writes(rows_b, b_c + (2 * g + 1) * bpb)
                return 0

            lax.fori_loop(0, nb // 2, pair_body, 0)
            pltpu.make_async_copy(
                table_h.at[idx_v.at[pl.ds(0, BLK)]], rows_a, sem_a).wait()

    return k(table, action2)


def kernel(action, rule_table, token_table):
    V = rule_table.shape[0]
    _, B, L = action.shape
    N = B * L
    table = _pad_stack_tables(rule_table, token_table)
    action2 = action.reshape(2, N)
    out = _action_embed(B, L, V, table, action2)
    return out[:, :L, :D]
